# baseline (device time: 261614 ns/iter reference)
import jax
import jax.numpy as jnp
from jax import lax
from jax.experimental import pallas as pl
from jax.experimental.pallas import tpu as pltpu

N_DEV = 16
N_TOK = 2048
D_IN = 512
D_OUT = 1024
N_EXP = 128
EXP_PER_DEV = N_EXP // N_DEV
CAP = 12
CHUNK = N_TOK // N_DEV


def kernel(x, router_W, route_idx, expert_W):
    del router_W

    r = route_idx[:, 0]
    oh = r[:, None] == jnp.arange(N_EXP, dtype=r.dtype)[None, :]
    pos = jnp.cumsum(oh.astype(jnp.int32), axis=0)
    keep = (oh & (pos <= CAP)).astype(jnp.bfloat16)

    my = lax.axis_index("i")
    keep_local = lax.dynamic_slice(
        keep, (0, my * EXP_PER_DEV), (N_TOK, EXP_PER_DEV)
    )

    x16 = x.astype(jnp.bfloat16)
    w16 = expert_W.astype(jnp.bfloat16)

    def body(x_ref, k_ref, w_ref, out_ref, rs_buf, rs_send, rs_recv, ag_send, ag_recv):
        my_pos = lax.axis_index("i")
        left = lax.rem(my_pos + N_DEV - 1, N_DEV)
        right = lax.rem(my_pos + 1, N_DEV)

        barrier_sem = pltpu.get_barrier_semaphore()
        for nbr in (left, right):
            pl.semaphore_signal(
                barrier_sem, inc=1,
                device_id=(nbr,), device_id_type=pl.DeviceIdType.MESH,
            )
        pl.semaphore_wait(barrier_sem, 2)

        acc = jnp.zeros((N_TOK, D_OUT), jnp.float32)
        for le in range(EXP_PER_DEV):
            xm = x_ref[:, :] * k_ref[:, le : le + 1]
            acc = acc + jnp.dot(xm, w_ref[le], preferred_element_type=jnp.float32)
        out_ref[:, :] = acc

        rs_buf[0, :, :] = out_ref[pl.ds(my_pos * CHUNK, CHUNK), :]
        for h in range(N_DEV - 1):
            rdma = pltpu.make_async_remote_copy(
                src_ref=rs_buf.at[h],
                dst_ref=rs_buf.at[h + 1],
                send_sem=rs_send.at[h],
                recv_sem=rs_recv.at[h + 1],
                device_id=(right,),
                device_id_type=pl.DeviceIdType.MESH,
            )
            rdma.start()
            rdma.wait()
            c = lax.rem(my_pos + N_DEV - h - 1, N_DEV)
            rs_buf[h + 1, :, :] = (
                rs_buf[h + 1, :, :] + out_ref[pl.ds(c * CHUNK, CHUNK), :]
            )
        own = lax.rem(my_pos + 1, N_DEV)
        out_ref[pl.ds(own * CHUNK, CHUNK), :] = rs_buf[N_DEV - 1, :, :]

        for s in range(N_DEV - 1):
            c = lax.rem(my_pos + 1 - s + N_DEV, N_DEV)
            rdma = pltpu.make_async_remote_copy(
                src_ref=out_ref.at[pl.ds(c * CHUNK, CHUNK), :],
                dst_ref=out_ref.at[pl.ds(c * CHUNK, CHUNK), :],
                send_sem=ag_send.at[s],
                recv_sem=ag_recv.at[s],
                device_id=(right,),
                device_id_type=pl.DeviceIdType.MESH,
            )
            rdma.start()
            rdma.wait()

    return pl.pallas_call(
        body,
        out_shape=jax.ShapeDtypeStruct((N_TOK, D_OUT), jnp.float32),
        in_specs=[
            pl.BlockSpec(memory_space=pltpu.VMEM),
            pl.BlockSpec(memory_space=pltpu.VMEM),
            pl.BlockSpec(memory_space=pltpu.VMEM),
        ],
        out_specs=pl.BlockSpec(memory_space=pltpu.VMEM),
        scratch_shapes=[
            pltpu.VMEM((N_DEV, CHUNK, D_OUT), jnp.float32),
            pltpu.SemaphoreType.DMA((N_DEV - 1,)),
            pltpu.SemaphoreType.DMA((N_DEV,)),
            pltpu.SemaphoreType.DMA((N_DEV - 1,)),
            pltpu.SemaphoreType.DMA((N_DEV - 1,)),
        ],
        compiler_params=pltpu.CompilerParams(collective_id=0),
    )(x16, keep_local, w16)


# device time: 169753 ns/iter; 1.5411x vs baseline; 1.5411x over previous
import jax
import jax.numpy as jnp
from jax import lax
from jax.experimental import pallas as pl
from jax.experimental.pallas import tpu as pltpu

N_DEV = 16
N_TOK = 2048
D_IN = 512
D_OUT = 1024
HALF = D_OUT // 2
N_EXP = 128
EXP_PER_DEV = N_EXP // N_DEV
CAP = 12
CHUNK = N_TOK // N_DEV


def kernel(x, router_W, route_idx, expert_W):
    del router_W

    r = route_idx[:, 0]
    oh = r[:, None] == jnp.arange(N_EXP, dtype=r.dtype)[None, :]
    pos = jnp.cumsum(oh.astype(jnp.int32), axis=0)
    keep = (oh & (pos <= CAP)).astype(jnp.bfloat16)

    my = lax.axis_index("i")
    keep_local = lax.dynamic_slice(
        keep, (0, my * EXP_PER_DEV), (N_TOK, EXP_PER_DEV)
    )

    x16 = x.astype(jnp.bfloat16)
    w16 = expert_W.astype(jnp.bfloat16)

    def body(
        x_ref, k_ref, w_ref, out_ref,
        cw_buf, ccw_buf,
        cw_rs_send, cw_rs_recv, cw_ag_send, cw_ag_recv,
        ccw_rs_send, ccw_rs_recv, ccw_ag_send, ccw_ag_recv,
    ):
        my_pos = lax.axis_index("i")
        left = lax.rem(my_pos + N_DEV - 1, N_DEV)
        right = lax.rem(my_pos + 1, N_DEV)

        barrier_sem = pltpu.get_barrier_semaphore()
        for nbr in (left, right):
            pl.semaphore_signal(
                barrier_sem, inc=1,
                device_id=(nbr,), device_id_type=pl.DeviceIdType.MESH,
            )
        pl.semaphore_wait(barrier_sem, 2)

        def rs_hop(direction, h):
            buf = cw_buf if direction == 0 else ccw_buf
            rdma = pltpu.make_async_remote_copy(
                src_ref=buf.at[h],
                dst_ref=buf.at[h + 1],
                send_sem=(cw_rs_send if direction == 0 else ccw_rs_send).at[h],
                recv_sem=(cw_rs_recv if direction == 0 else ccw_rs_recv).at[h + 1],
                device_id=(right if direction == 0 else left,),
                device_id_type=pl.DeviceIdType.MESH,
            )
            return rdma

        rows_me = pl.ds(my_pos * CHUNK, CHUNK)
        acc0 = jnp.zeros((CHUNK, D_OUT), jnp.float32)
        for le in range(EXP_PER_DEV):
            xm = x_ref[rows_me, :] * k_ref[rows_me, le : le + 1]
            acc0 = acc0 + jnp.dot(xm, w_ref[le], preferred_element_type=jnp.float32)
        own16 = acc0.astype(jnp.bfloat16)
        cw_buf[0, :, :] = own16[:, :HALF]
        ccw_buf[0, :, :] = own16[:, HALF:]
        cw0 = rs_hop(0, 0)
        ccw0 = rs_hop(1, 0)
        cw0.start()
        ccw0.start()

        acc = jnp.zeros((N_TOK, D_OUT), jnp.float32)
        for le in range(EXP_PER_DEV):
            xm = x_ref[:, :] * k_ref[:, le : le + 1]
            acc = acc + jnp.dot(xm, w_ref[le], preferred_element_type=jnp.float32)
        out_ref[:, :] = acc.astype(jnp.bfloat16)

        pending = (cw0, ccw0)
        for h in range(N_DEV - 1):
            pending[0].wait()
            pending[1].wait()
            c_cw = lax.rem(my_pos + N_DEV - h - 1, N_DEV)
            c_ccw = lax.rem(my_pos + h + 1, N_DEV)
            cw_buf[h + 1, :, :] = (
                cw_buf[h + 1, :, :]
                + out_ref[pl.ds(c_cw * CHUNK, CHUNK), pl.ds(0, HALF)]
            )
            ccw_buf[h + 1, :, :] = (
                ccw_buf[h + 1, :, :]
                + out_ref[pl.ds(c_ccw * CHUNK, CHUNK), pl.ds(HALF, HALF)]
            )
            if h < N_DEV - 2:
                cw_n = rs_hop(0, h + 1)
                ccw_n = rs_hop(1, h + 1)
                cw_n.start()
                ccw_n.start()
                pending = (cw_n, ccw_n)

        own_cw = lax.rem(my_pos + 1, N_DEV)
        own_ccw = left
        out_ref[pl.ds(own_cw * CHUNK, CHUNK), pl.ds(0, HALF)] = cw_buf[N_DEV - 1]
        out_ref[pl.ds(own_ccw * CHUNK, CHUNK), pl.ds(HALF, HALF)] = ccw_buf[N_DEV - 1]

        for s in range(N_DEV - 1):
            c1 = lax.rem(my_pos + 1 - s + N_DEV, N_DEV)
            sl1 = (pl.ds(c1 * CHUNK, CHUNK), pl.ds(0, HALF))
            ag_cw = pltpu.make_async_remote_copy(
                src_ref=out_ref.at[sl1],
                dst_ref=out_ref.at[sl1],
                send_sem=cw_ag_send.at[s],
                recv_sem=cw_ag_recv.at[s],
                device_id=(right,),
                device_id_type=pl.DeviceIdType.MESH,
            )
            c2 = lax.rem(my_pos + N_DEV - 1 + s, N_DEV)
            sl2 = (pl.ds(c2 * CHUNK, CHUNK), pl.ds(HALF, HALF))
            ag_ccw = pltpu.make_async_remote_copy(
                src_ref=out_ref.at[sl2],
                dst_ref=out_ref.at[sl2],
                send_sem=ccw_ag_send.at[s],
                recv_sem=ccw_ag_recv.at[s],
                device_id=(left,),
                device_id_type=pl.DeviceIdType.MESH,
            )
            ag_cw.start()
            ag_ccw.start()
            ag_cw.wait()
            ag_ccw.wait()

    return pl.pallas_call(
        body,
        out_shape=jax.ShapeDtypeStruct((N_TOK, D_OUT), jnp.bfloat16),
        in_specs=[
            pl.BlockSpec(memory_space=pltpu.VMEM),
            pl.BlockSpec(memory_space=pltpu.VMEM),
            pl.BlockSpec(memory_space=pltpu.VMEM),
        ],
        out_specs=pl.BlockSpec(memory_space=pltpu.VMEM),
        scratch_shapes=[
            pltpu.VMEM((N_DEV, CHUNK, HALF), jnp.bfloat16),
            pltpu.VMEM((N_DEV, CHUNK, HALF), jnp.bfloat16),
            pltpu.SemaphoreType.DMA((N_DEV - 1,)),
            pltpu.SemaphoreType.DMA((N_DEV,)),
            pltpu.SemaphoreType.DMA((N_DEV - 1,)),
            pltpu.SemaphoreType.DMA((N_DEV - 1,)),
            pltpu.SemaphoreType.DMA((N_DEV - 1,)),
            pltpu.SemaphoreType.DMA((N_DEV,)),
            pltpu.SemaphoreType.DMA((N_DEV - 1,)),
            pltpu.SemaphoreType.DMA((N_DEV - 1,)),
        ],
        compiler_params=pltpu.CompilerParams(collective_id=0),
    )(x16, keep_local, w16)


# device time: 101282 ns/iter; 2.5830x vs baseline; 1.6760x over previous
import jax
import jax.numpy as jnp
from jax import lax
from jax.experimental import pallas as pl
from jax.experimental.pallas import tpu as pltpu

N_DEV = 16
N_TOK = 2048
D_IN = 512
D_OUT = 1024
HALF = D_OUT // 2
N_EXP = 128
EXP_PER_DEV = N_EXP // N_DEV
CAP = 12
SLOT = 16
CROWS = EXP_PER_DEV * SLOT
N_SLOTS = N_DEV * CROWS


def kernel(x, router_W, route_idx, expert_W):
    del router_W

    r = route_idx[:, 0].astype(jnp.int32)
    oh = r[:, None] == jnp.arange(N_EXP, dtype=jnp.int32)[None, :]
    pos = jnp.cumsum(oh.astype(jnp.int32), axis=0)
    rank = jnp.take_along_axis(pos, r[:, None], axis=1)[:, 0] - 1
    kept = rank < CAP
    gslot = jnp.where(kept, (r // EXP_PER_DEV) * CROWS + (r % EXP_PER_DEV) * SLOT + rank, -1)

    my = lax.axis_index("i")
    G_my = (
        jnp.arange(CROWS, dtype=jnp.int32)[:, None] == (gslot[None, :] - my * CROWS)
    ).astype(jnp.bfloat16)
    P = (gslot[:, None] == jnp.arange(N_SLOTS, dtype=jnp.int32)[None, :]).astype(
        jnp.bfloat16
    )

    x16 = x.astype(jnp.bfloat16)
    w16 = expert_W.astype(jnp.bfloat16)

    def body(
        x_ref, g_ref, p_ref, w_ref, out_ref, cbuf,
        cw_send, cw_recv, ccw_send, ccw_recv,
    ):
        my_pos = lax.axis_index("i")
        left = lax.rem(my_pos + N_DEV - 1, N_DEV)
        right = lax.rem(my_pos + 1, N_DEV)

        barrier_sem = pltpu.get_barrier_semaphore()
        for nbr in (left, right):
            pl.semaphore_signal(
                barrier_sem, inc=1,
                device_id=(nbr,), device_id_type=pl.DeviceIdType.MESH,
            )
        pl.semaphore_wait(barrier_sem, 2)

        xg = jnp.dot(
            g_ref[:, :], x_ref[:, :], preferred_element_type=jnp.float32
        ).astype(jnp.bfloat16)
        rows_me = pl.ds(my_pos * CROWS, CROWS)
        for le in range(EXP_PER_DEV):
            cbuf[
                pl.ds(my_pos * CROWS + le * SLOT, SLOT), :
            ] = jnp.dot(
                xg[le * SLOT : (le + 1) * SLOT, :],
                w_ref[le],
                preferred_element_type=jnp.float32,
            ).astype(jnp.bfloat16)

        def hop(s):
            c1 = lax.rem(my_pos - s + N_DEV, N_DEV)
            sl1 = (pl.ds(c1 * CROWS, CROWS), pl.ds(0, HALF))
            cw = pltpu.make_async_remote_copy(
                src_ref=cbuf.at[sl1], dst_ref=cbuf.at[sl1],
                send_sem=cw_send.at[s], recv_sem=cw_recv.at[s],
                device_id=(right,), device_id_type=pl.DeviceIdType.MESH,
            )
            c2 = lax.rem(my_pos + s, N_DEV)
            sl2 = (pl.ds(c2 * CROWS, CROWS), pl.ds(HALF, HALF))
            ccw = pltpu.make_async_remote_copy(
                src_ref=cbuf.at[sl2], dst_ref=cbuf.at[sl2],
                send_sem=ccw_send.at[s], recv_sem=ccw_recv.at[s],
                device_id=(left,), device_id_type=pl.DeviceIdType.MESH,
            )
            cw.start()
            ccw.start()
            return cw, ccw

        pending = hop(0)

        sl_lo = pl.ds(0, HALF)
        sl_hi = pl.ds(HALF, HALF)
        out_ref[:, sl_lo] = jnp.dot(
            p_ref[:, rows_me], cbuf[rows_me, sl_lo],
            preferred_element_type=jnp.float32,
        ).astype(jnp.bfloat16)
        out_ref[:, sl_hi] = jnp.dot(
            p_ref[:, rows_me], cbuf[rows_me, sl_hi],
            preferred_element_type=jnp.float32,
        ).astype(jnp.bfloat16)

        for s in range(N_DEV - 1):
            pending[0].wait()
            pending[1].wait()
            if s < N_DEV - 2:
                pending = hop(s + 1)
            k1 = lax.rem(my_pos - 1 - s + N_DEV, N_DEV)
            k2 = lax.rem(my_pos + 1 + s, N_DEV)
            out_ref[:, sl_lo] = out_ref[:, sl_lo] + jnp.dot(
                p_ref[:, pl.ds(k1 * CROWS, CROWS)],
                cbuf[pl.ds(k1 * CROWS, CROWS), sl_lo],
                preferred_element_type=jnp.float32,
            ).astype(jnp.bfloat16)
            out_ref[:, sl_hi] = out_ref[:, sl_hi] + jnp.dot(
                p_ref[:, pl.ds(k2 * CROWS, CROWS)],
                cbuf[pl.ds(k2 * CROWS, CROWS), sl_hi],
                preferred_element_type=jnp.float32,
            ).astype(jnp.bfloat16)

    return pl.pallas_call(
        body,
        out_shape=jax.ShapeDtypeStruct((N_TOK, D_OUT), jnp.bfloat16),
        in_specs=[
            pl.BlockSpec(memory_space=pltpu.VMEM),
            pl.BlockSpec(memory_space=pltpu.VMEM),
            pl.BlockSpec(memory_space=pltpu.VMEM),
            pl.BlockSpec(memory_space=pltpu.VMEM),
        ],
        out_specs=pl.BlockSpec(memory_space=pltpu.VMEM),
        scratch_shapes=[
            pltpu.VMEM((N_SLOTS, D_OUT), jnp.bfloat16),
            pltpu.SemaphoreType.DMA((N_DEV - 1,)),
            pltpu.SemaphoreType.DMA((N_DEV - 1,)),
            pltpu.SemaphoreType.DMA((N_DEV - 1,)),
            pltpu.SemaphoreType.DMA((N_DEV - 1,)),
        ],
        compiler_params=pltpu.CompilerParams(collective_id=0),
    )(x16, G_my, P, w16)


# device time: 94629 ns/iter; 2.7646x vs baseline; 1.0703x over previous
import jax
import jax.numpy as jnp
from jax import lax
from jax.experimental import pallas as pl
from jax.experimental.pallas import tpu as pltpu

N_DEV = 16
N_TOK = 2048
D_IN = 512
D_OUT = 1024
HALF = D_OUT // 2
N_EXP = 128
EXP_PER_DEV = N_EXP // N_DEV
CAP = 12
SLOT = 16
CROWS = EXP_PER_DEV * SLOT
N_SLOTS = N_DEV * CROWS


def kernel(x, router_W, route_idx, expert_W):
    del router_W

    r = route_idx[:, 0].astype(jnp.int32)
    oh = r[:, None] == jnp.arange(N_EXP, dtype=jnp.int32)[None, :]
    pos = jnp.cumsum(oh.astype(jnp.int32), axis=0)
    rank = jnp.take_along_axis(pos, r[:, None], axis=1)[:, 0] - 1
    kept = rank < CAP
    gslot = jnp.where(
        kept, (r // EXP_PER_DEV) * CROWS + (r % EXP_PER_DEV) * SLOT + rank, -1
    )
    gslot_row = gslot[None, :]
    gslot_col = gslot[:, None]

    x16 = x.astype(jnp.bfloat16)
    w16 = expert_W.astype(jnp.bfloat16)

    def body(
        x_ref, gr_ref, gc_ref, w_ref, out_ref, cbuf,
        cw_send, cw_recv, ccw_send, ccw_recv,
    ):
        my_pos = lax.axis_index("i")
        left = lax.rem(my_pos + N_DEV - 1, N_DEV)
        right = lax.rem(my_pos + 1, N_DEV)

        barrier_sem = pltpu.get_barrier_semaphore()
        for nbr in (left, right):
            pl.semaphore_signal(
                barrier_sem, inc=1,
                device_id=(nbr,), device_id_type=pl.DeviceIdType.MESH,
            )
        pl.semaphore_wait(barrier_sem, 2)

        g_my = (
            lax.broadcasted_iota(jnp.int32, (CROWS, N_TOK), 0)
            == (gr_ref[:, :] - my_pos * CROWS)
        ).astype(jnp.bfloat16)
        xg = jnp.dot(
            g_my, x_ref[:, :], preferred_element_type=jnp.float32
        ).astype(jnp.bfloat16)
        for le in range(EXP_PER_DEV):
            cbuf[
                pl.ds(my_pos * CROWS + le * SLOT, SLOT), :
            ] = jnp.dot(
                xg[le * SLOT : (le + 1) * SLOT, :],
                w_ref[le],
                preferred_element_type=jnp.float32,
            ).astype(jnp.bfloat16)

        def hop(s):
            c1 = lax.rem(my_pos - s + N_DEV, N_DEV)
            sl1 = (pl.ds(c1 * CROWS, CROWS), pl.ds(0, HALF))
            cw = pltpu.make_async_remote_copy(
                src_ref=cbuf.at[sl1], dst_ref=cbuf.at[sl1],
                send_sem=cw_send.at[s], recv_sem=cw_recv.at[s],
                device_id=(right,), device_id_type=pl.DeviceIdType.MESH,
            )
            c2 = lax.rem(my_pos + s, N_DEV)
            sl2 = (pl.ds(c2 * CROWS, CROWS), pl.ds(HALF, HALF))
            ccw = pltpu.make_async_remote_copy(
                src_ref=cbuf.at[sl2], dst_ref=cbuf.at[sl2],
                send_sem=ccw_send.at[s], recv_sem=ccw_recv.at[s],
                device_id=(left,), device_id_type=pl.DeviceIdType.MESH,
            )
            cw.start()
            ccw.start()
            return cw, ccw

        pending = hop(0)

        def pblock(k):
            return (
                gc_ref[:, :]
                == lax.broadcasted_iota(jnp.int32, (N_TOK, CROWS), 1) + k * CROWS
            ).astype(jnp.bfloat16)

        sl_lo = pl.ds(0, HALF)
        sl_hi = pl.ds(HALF, HALF)
        rows_me = pl.ds(my_pos * CROWS, CROWS)
        p_me = pblock(my_pos)
        out_ref[:, sl_lo] = jnp.dot(
            p_me, cbuf[rows_me, sl_lo], preferred_element_type=jnp.float32
        ).astype(jnp.bfloat16)
        out_ref[:, sl_hi] = jnp.dot(
            p_me, cbuf[rows_me, sl_hi], preferred_element_type=jnp.float32
        ).astype(jnp.bfloat16)

        for s in range(N_DEV - 1):
            pending[0].wait()
            pending[1].wait()
            if s < N_DEV - 2:
                pending = hop(s + 1)
            k1 = lax.rem(my_pos - 1 - s + N_DEV, N_DEV)
            k2 = lax.rem(my_pos + 1 + s, N_DEV)
            out_ref[:, sl_lo] = out_ref[:, sl_lo] + jnp.dot(
                pblock(k1),
                cbuf[pl.ds(k1 * CROWS, CROWS), sl_lo],
                preferred_element_type=jnp.float32,
            ).astype(jnp.bfloat16)
            out_ref[:, sl_hi] = out_ref[:, sl_hi] + jnp.dot(
                pblock(k2),
                cbuf[pl.ds(k2 * CROWS, CROWS), sl_hi],
                preferred_element_type=jnp.float32,
            ).astype(jnp.bfloat16)

    return pl.pallas_call(
        body,
        out_shape=jax.ShapeDtypeStruct((N_TOK, D_OUT), jnp.bfloat16),
        in_specs=[
            pl.BlockSpec(memory_space=pltpu.VMEM),
            pl.BlockSpec(memory_space=pltpu.VMEM),
            pl.BlockSpec(memory_space=pltpu.VMEM),
            pl.BlockSpec(memory_space=pltpu.VMEM),
        ],
        out_specs=pl.BlockSpec(memory_space=pltpu.VMEM),
        scratch_shapes=[
            pltpu.VMEM((N_SLOTS, D_OUT), jnp.bfloat16),
            pltpu.SemaphoreType.DMA((N_DEV - 1,)),
            pltpu.SemaphoreType.DMA((N_DEV - 1,)),
            pltpu.SemaphoreType.DMA((N_DEV - 1,)),
            pltpu.SemaphoreType.DMA((N_DEV - 1,)),
        ],
        compiler_params=pltpu.CompilerParams(collective_id=0),
    )(x16, gslot_row, gslot_col, w16)


# device time: 94619 ns/iter; 2.7649x vs baseline; 1.0001x over previous
import jax
import jax.numpy as jnp
from jax import lax
from jax.experimental import pallas as pl
from jax.experimental.pallas import tpu as pltpu

N_DEV = 16
N_TOK = 2048
D_IN = 512
D_OUT = 1024
HALF = D_OUT // 2
N_EXP = 128
EXP_PER_DEV = N_EXP // N_DEV
CAP = 12
SLOT = 16
CROWS = EXP_PER_DEV * SLOT
N_SLOTS = N_DEV * CROWS


def kernel(x, router_W, route_idx, expert_W):
    del router_W

    x16 = x.astype(jnp.bfloat16)
    w16 = expert_W.astype(jnp.bfloat16)
    r_col = route_idx.astype(jnp.int32)

    def body(
        x_ref, r_ref, w_ref, out_ref, cbuf,
        cw_send, cw_recv, ccw_send, ccw_recv,
    ):
        my_pos = lax.axis_index("i")
        left = lax.rem(my_pos + N_DEV - 1, N_DEV)
        right = lax.rem(my_pos + 1, N_DEV)

        barrier_sem = pltpu.get_barrier_semaphore()
        for nbr in (left, right):
            pl.semaphore_signal(
                barrier_sem, inc=1,
                device_id=(nbr,), device_id_type=pl.DeviceIdType.MESH,
            )
        pl.semaphore_wait(barrier_sem, 2)

        ri = r_ref[:, :]
        ohb = ri == lax.broadcasted_iota(jnp.int32, (N_TOK, N_EXP), 1)
        oh16 = ohb.astype(jnp.bfloat16)
        tri = (
            lax.broadcasted_iota(jnp.int32, (N_TOK, N_TOK), 0)
            >= lax.broadcasted_iota(jnp.int32, (N_TOK, N_TOK), 1)
        ).astype(jnp.bfloat16)
        pos = jnp.dot(tri, oh16, preferred_element_type=jnp.float32)
        rank = (
            jnp.sum(jnp.where(ohb, pos, 0.0), axis=1, keepdims=True) - 1.0
        )
        kept = rank < CAP
        gsl = jnp.where(
            kept,
            (ri // EXP_PER_DEV) * CROWS + (ri % EXP_PER_DEV) * SLOT
            + rank.astype(jnp.int32),
            -1,
        )

        def pblock(k):
            return (
                gsl
                == lax.broadcasted_iota(jnp.int32, (N_TOK, CROWS), 1) + k * CROWS
            ).astype(jnp.bfloat16)

        p_me = pblock(my_pos)
        xg = lax.dot_general(
            p_me, x_ref[:, :],
            dimension_numbers=(((0,), (0,)), ((), ())),
            preferred_element_type=jnp.float32,
        ).astype(jnp.bfloat16)
        for le in range(EXP_PER_DEV):
            cbuf[
                pl.ds(my_pos * CROWS + le * SLOT, SLOT), :
            ] = jnp.dot(
                xg[le * SLOT : (le + 1) * SLOT, :],
                w_ref[le],
                preferred_element_type=jnp.float32,
            ).astype(jnp.bfloat16)

        def hop(s):
            c1 = lax.rem(my_pos - s + N_DEV, N_DEV)
            sl1 = (pl.ds(c1 * CROWS, CROWS), pl.ds(0, HALF))
            cw = pltpu.make_async_remote_copy(
                src_ref=cbuf.at[sl1], dst_ref=cbuf.at[sl1],
                send_sem=cw_send.at[s], recv_sem=cw_recv.at[s],
                device_id=(right,), device_id_type=pl.DeviceIdType.MESH,
            )
            c2 = lax.rem(my_pos + s, N_DEV)
            sl2 = (pl.ds(c2 * CROWS, CROWS), pl.ds(HALF, HALF))
            ccw = pltpu.make_async_remote_copy(
                src_ref=cbuf.at[sl2], dst_ref=cbuf.at[sl2],
                send_sem=ccw_send.at[s], recv_sem=ccw_recv.at[s],
                device_id=(left,), device_id_type=pl.DeviceIdType.MESH,
            )
            cw.start()
            ccw.start()
            return cw, ccw

        pending = hop(0)

        sl_lo = pl.ds(0, HALF)
        sl_hi = pl.ds(HALF, HALF)
        rows_me = pl.ds(my_pos * CROWS, CROWS)
        out_ref[:, sl_lo] = jnp.dot(
            p_me, cbuf[rows_me, sl_lo], preferred_element_type=jnp.float32
        ).astype(jnp.bfloat16)
        out_ref[:, sl_hi] = jnp.dot(
            p_me, cbuf[rows_me, sl_hi], preferred_element_type=jnp.float32
        ).astype(jnp.bfloat16)

        for s in range(N_DEV - 1):
            pending[0].wait()
            pending[1].wait()
            if s < N_DEV - 2:
                pending = hop(s + 1)
            k1 = lax.rem(my_pos - 1 - s + N_DEV, N_DEV)
            k2 = lax.rem(my_pos + 1 + s, N_DEV)
            out_ref[:, sl_lo] = out_ref[:, sl_lo] + jnp.dot(
                pblock(k1),
                cbuf[pl.ds(k1 * CROWS, CROWS), sl_lo],
                preferred_element_type=jnp.float32,
            ).astype(jnp.bfloat16)
            out_ref[:, sl_hi] = out_ref[:, sl_hi] + jnp.dot(
                pblock(k2),
                cbuf[pl.ds(k2 * CROWS, CROWS), sl_hi],
                preferred_element_type=jnp.float32,
            ).astype(jnp.bfloat16)

    return pl.pallas_call(
        body,
        out_shape=jax.ShapeDtypeStruct((N_TOK, D_OUT), jnp.bfloat16),
        in_specs=[
            pl.BlockSpec(memory_space=pltpu.VMEM),
            pl.BlockSpec(memory_space=pltpu.VMEM),
            pl.BlockSpec(memory_space=pltpu.VMEM),
        ],
        out_specs=pl.BlockSpec(memory_space=pltpu.VMEM),
        scratch_shapes=[
            pltpu.VMEM((N_SLOTS, D_OUT), jnp.bfloat16),
            pltpu.SemaphoreType.DMA((N_DEV - 1,)),
            pltpu.SemaphoreType.DMA((N_DEV - 1,)),
            pltpu.SemaphoreType.DMA((N_DEV - 1,)),
            pltpu.SemaphoreType.DMA((N_DEV - 1,)),
        ],
        compiler_params=pltpu.CompilerParams(collective_id=0),
    )(x16, r_col, w16)


# device time: 87745 ns/iter; 2.9815x vs baseline; 1.0783x over previous
import jax
import jax.numpy as jnp
from jax import lax
from jax.experimental import pallas as pl
from jax.experimental.pallas import tpu as pltpu

N_DEV = 16
N_TOK = 2048
D_IN = 512
D_OUT = 1024
HALF = D_OUT // 2
N_EXP = 128
EXP_PER_DEV = N_EXP // N_DEV
CAP = 12
SLOT = 16
CROWS = EXP_PER_DEV * SLOT
N_SLOTS = N_DEV * CROWS


def kernel(x, router_W, route_idx, expert_W):
    del router_W

    x16 = x.astype(jnp.bfloat16)
    w16 = expert_W.astype(jnp.bfloat16)
    r_col = route_idx.astype(jnp.int32)

    def body(
        x_ref, r_ref, w_ref, out_ref, cbuf,
        cw_send, cw_recv, ccw_send, ccw_recv,
    ):
        my_pos = lax.axis_index("i")
        left = lax.rem(my_pos + N_DEV - 1, N_DEV)
        right = lax.rem(my_pos + 1, N_DEV)

        barrier_sem = pltpu.get_barrier_semaphore()
        for nbr in (left, right):
            pl.semaphore_signal(
                barrier_sem, inc=1,
                device_id=(nbr,), device_id_type=pl.DeviceIdType.MESH,
            )
        pl.semaphore_wait(barrier_sem, 2)

        ri = r_ref[:, :]
        ohb = ri == lax.broadcasted_iota(jnp.int32, (N_TOK, N_EXP), 1)
        oh16 = ohb.astype(jnp.bfloat16)
        BLK = 128
        n_blk = N_TOK // BLK
        tri = (
            lax.broadcasted_iota(jnp.int32, (BLK, BLK), 0)
            >= lax.broadcasted_iota(jnp.int32, (BLK, BLK), 1)
        ).astype(jnp.bfloat16)
        pos_w = []
        blk_tot = []
        for b in range(n_blk):
            blk = oh16[b * BLK : (b + 1) * BLK, :]
            pw = jnp.dot(tri, blk, preferred_element_type=jnp.float32)
            pos_w.append(pw)
            blk_tot.append(pw[BLK - 1 : BLK, :])
        ranks = []
        off = jnp.zeros((1, N_EXP), jnp.float32)
        for b in range(n_blk):
            pos_b = pos_w[b] + off
            ohb_b = ohb[b * BLK : (b + 1) * BLK, :]
            ranks.append(
                jnp.sum(jnp.where(ohb_b, pos_b, 0.0), axis=1, keepdims=True)
            )
            off = jnp.minimum(off + blk_tot[b], 2.0 * SLOT)
        rank = jnp.concatenate(ranks, axis=0) - 1.0
        kept = rank < CAP
        gsl = jnp.where(
            kept,
            (ri // EXP_PER_DEV) * CROWS + (ri % EXP_PER_DEV) * SLOT
            + rank.astype(jnp.int32),
            -1,
        )

        def pblock(k):
            return (
                gsl
                == lax.broadcasted_iota(jnp.int32, (N_TOK, CROWS), 1) + k * CROWS
            ).astype(jnp.bfloat16)

        p_me = pblock(my_pos)
        xg = lax.dot_general(
            p_me, x_ref[:, :],
            dimension_numbers=(((0,), (0,)), ((), ())),
            preferred_element_type=jnp.float32,
        ).astype(jnp.bfloat16)
        for le in range(EXP_PER_DEV):
            cbuf[
                pl.ds(my_pos * CROWS + le * SLOT, SLOT), :
            ] = jnp.dot(
                xg[le * SLOT : (le + 1) * SLOT, :],
                w_ref[le],
                preferred_element_type=jnp.float32,
            ).astype(jnp.bfloat16)

        def hop(s):
            c1 = lax.rem(my_pos - s + N_DEV, N_DEV)
            sl1 = (pl.ds(c1 * CROWS, CROWS), pl.ds(0, HALF))
            cw = pltpu.make_async_remote_copy(
                src_ref=cbuf.at[sl1], dst_ref=cbuf.at[sl1],
                send_sem=cw_send.at[s], recv_sem=cw_recv.at[s],
                device_id=(right,), device_id_type=pl.DeviceIdType.MESH,
            )
            c2 = lax.rem(my_pos + s, N_DEV)
            sl2 = (pl.ds(c2 * CROWS, CROWS), pl.ds(HALF, HALF))
            ccw = pltpu.make_async_remote_copy(
                src_ref=cbuf.at[sl2], dst_ref=cbuf.at[sl2],
                send_sem=ccw_send.at[s], recv_sem=ccw_recv.at[s],
                device_id=(left,), device_id_type=pl.DeviceIdType.MESH,
            )
            cw.start()
            ccw.start()
            return cw, ccw

        pending = hop(0)

        sl_lo = pl.ds(0, HALF)
        sl_hi = pl.ds(HALF, HALF)
        rows_me = pl.ds(my_pos * CROWS, CROWS)
        out_ref[:, sl_lo] = jnp.dot(
            p_me, cbuf[rows_me, sl_lo], preferred_element_type=jnp.float32
        ).astype(jnp.bfloat16)
        out_ref[:, sl_hi] = jnp.dot(
            p_me, cbuf[rows_me, sl_hi], preferred_element_type=jnp.float32
        ).astype(jnp.bfloat16)

        for s in range(N_DEV - 1):
            pending[0].wait()
            pending[1].wait()
            if s < N_DEV - 2:
                pending = hop(s + 1)
            k1 = lax.rem(my_pos - 1 - s + N_DEV, N_DEV)
            k2 = lax.rem(my_pos + 1 + s, N_DEV)
            out_ref[:, sl_lo] = out_ref[:, sl_lo] + jnp.dot(
                pblock(k1),
                cbuf[pl.ds(k1 * CROWS, CROWS), sl_lo],
                preferred_element_type=jnp.float32,
            ).astype(jnp.bfloat16)
            out_ref[:, sl_hi] = out_ref[:, sl_hi] + jnp.dot(
                pblock(k2),
                cbuf[pl.ds(k2 * CROWS, CROWS), sl_hi],
                preferred_element_type=jnp.float32,
            ).astype(jnp.bfloat16)

    return pl.pallas_call(
        body,
        out_shape=jax.ShapeDtypeStruct((N_TOK, D_OUT), jnp.bfloat16),
        in_specs=[
            pl.BlockSpec(memory_space=pltpu.VMEM),
            pl.BlockSpec(memory_space=pltpu.VMEM),
            pl.BlockSpec(memory_space=pltpu.VMEM),
        ],
        out_specs=pl.BlockSpec(memory_space=pltpu.VMEM),
        scratch_shapes=[
            pltpu.VMEM((N_SLOTS, D_OUT), jnp.bfloat16),
            pltpu.SemaphoreType.DMA((N_DEV - 1,)),
            pltpu.SemaphoreType.DMA((N_DEV - 1,)),
            pltpu.SemaphoreType.DMA((N_DEV - 1,)),
            pltpu.SemaphoreType.DMA((N_DEV - 1,)),
        ],
        compiler_params=pltpu.CompilerParams(collective_id=0),
    )(x16, r_col, w16)


# device time: 67517 ns/iter; 3.8748x vs baseline; 1.2996x over previous
import jax
import jax.numpy as jnp
from jax import lax
from jax.experimental import pallas as pl
from jax.experimental.pallas import tpu as pltpu

N_DEV = 16
N_TOK = 2048
D_IN = 512
D_OUT = 1024
HALF = D_OUT // 2
N_EXP = 128
EXP_PER_DEV = N_EXP // N_DEV
CAP = 12
SLOT = 16
CROWS = EXP_PER_DEV * SLOT
N_SLOTS = N_DEV * CROWS


def kernel(x, router_W, route_idx, expert_W):
    del router_W

    r_col = route_idx.astype(jnp.int32)

    RING_ORDER = (0, 4, 8, 12, 13, 9, 5, 1, 2, 6, 10, 14, 15, 11, 7, 3)
    INV_RING = tuple(RING_ORDER.index(d) for d in range(N_DEV))

    def _lut(idx, table):
        out = jnp.int32(table[0])
        for j in range(1, N_DEV):
            out = jnp.where(idx == j, jnp.int32(table[j]), out)
        return out

    def body(
        x_ref, r_ref, w_ref, out_ref, cbuf,
        cw_send, cw_recv, ccw_send, ccw_recv,
    ):
        dev = lax.axis_index("i")
        my_pos = _lut(dev, INV_RING)
        left = _lut(lax.rem(my_pos + N_DEV - 1, N_DEV), RING_ORDER)
        right = _lut(lax.rem(my_pos + 1, N_DEV), RING_ORDER)

        barrier_sem = pltpu.get_barrier_semaphore()
        for nbr in (left, right):
            pl.semaphore_signal(
                barrier_sem, inc=1,
                device_id=(nbr,), device_id_type=pl.DeviceIdType.MESH,
            )
        pl.semaphore_wait(barrier_sem, 2)

        ri = r_ref[:, :]
        ohb = ri == lax.broadcasted_iota(jnp.int32, (N_TOK, N_EXP), 1)
        oh16 = ohb.astype(jnp.bfloat16)
        BLK = 128
        n_blk = N_TOK // BLK
        tri = (
            lax.broadcasted_iota(jnp.int32, (BLK, BLK), 0)
            >= lax.broadcasted_iota(jnp.int32, (BLK, BLK), 1)
        ).astype(jnp.bfloat16)
        pos_w = []
        blk_tot = []
        for b in range(n_blk):
            blk = oh16[b * BLK : (b + 1) * BLK, :]
            pw = jnp.dot(tri, blk, preferred_element_type=jnp.float32)
            pos_w.append(pw)
            blk_tot.append(pw[BLK - 1 : BLK, :])
        ranks = []
        off = jnp.zeros((1, N_EXP), jnp.float32)
        for b in range(n_blk):
            pos_b = pos_w[b] + off
            ohb_b = ohb[b * BLK : (b + 1) * BLK, :]
            ranks.append(
                jnp.sum(jnp.where(ohb_b, pos_b, 0.0), axis=1, keepdims=True)
            )
            off = jnp.minimum(off + blk_tot[b], 2.0 * SLOT)
        rank = jnp.concatenate(ranks, axis=0) - 1.0
        kept = rank < CAP
        devvec = ri // EXP_PER_DEV
        rpvec = jnp.zeros_like(devvec)
        for d in range(N_DEV):
            rpvec = jnp.where(devvec == d, jnp.int32(INV_RING[d]), rpvec)
        gsl = jnp.where(
            kept,
            rpvec * CROWS + (ri % EXP_PER_DEV) * SLOT + rank.astype(jnp.int32),
            -1,
        )

        def pblock(k):
            return (
                gsl
                == lax.broadcasted_iota(jnp.int32, (N_TOK, CROWS), 1) + k * CROWS
            ).astype(jnp.bfloat16)

        p_me = pblock(my_pos)
        xg = lax.dot_general(
            p_me.astype(jnp.float32), x_ref[:, :],
            dimension_numbers=(((0,), (0,)), ((), ())),
            preferred_element_type=jnp.float32,
        )
        for le in range(EXP_PER_DEV):
            cbuf[
                pl.ds(my_pos * CROWS + le * SLOT, SLOT), :
            ] = jnp.dot(
                xg[le * SLOT : (le + 1) * SLOT, :],
                w_ref[le],
                preferred_element_type=jnp.float32,
            ).astype(jnp.bfloat16)

        def hop(s):
            c1 = lax.rem(my_pos - s + N_DEV, N_DEV)
            sl1 = (pl.ds(c1 * CROWS, CROWS), pl.ds(0, HALF))
            cw = pltpu.make_async_remote_copy(
                src_ref=cbuf.at[sl1], dst_ref=cbuf.at[sl1],
                send_sem=cw_send.at[s], recv_sem=cw_recv.at[s],
                device_id=(right,), device_id_type=pl.DeviceIdType.MESH,
            )
            c2 = lax.rem(my_pos + s, N_DEV)
            sl2 = (pl.ds(c2 * CROWS, CROWS), pl.ds(HALF, HALF))
            ccw = pltpu.make_async_remote_copy(
                src_ref=cbuf.at[sl2], dst_ref=cbuf.at[sl2],
                send_sem=ccw_send.at[s], recv_sem=ccw_recv.at[s],
                device_id=(left,), device_id_type=pl.DeviceIdType.MESH,
            )
            cw.start()
            ccw.start()
            return cw, ccw

        pending = hop(0)

        sl_lo = pl.ds(0, HALF)
        sl_hi = pl.ds(HALF, HALF)
        rows_me = pl.ds(my_pos * CROWS, CROWS)
        out_ref[:, sl_lo] = jnp.dot(
            p_me, cbuf[rows_me, sl_lo], preferred_element_type=jnp.float32
        ).astype(jnp.bfloat16)
        out_ref[:, sl_hi] = jnp.dot(
            p_me, cbuf[rows_me, sl_hi], preferred_element_type=jnp.float32
        ).astype(jnp.bfloat16)

        for s in range(N_DEV - 1):
            pending[0].wait()
            pending[1].wait()
            if s < N_DEV - 2:
                pending = hop(s + 1)
            k1 = lax.rem(my_pos - 1 - s + N_DEV, N_DEV)
            k2 = lax.rem(my_pos + 1 + s, N_DEV)
            out_ref[:, sl_lo] = out_ref[:, sl_lo] + jnp.dot(
                pblock(k1),
                cbuf[pl.ds(k1 * CROWS, CROWS), sl_lo],
                preferred_element_type=jnp.float32,
            ).astype(jnp.bfloat16)
            out_ref[:, sl_hi] = out_ref[:, sl_hi] + jnp.dot(
                pblock(k2),
                cbuf[pl.ds(k2 * CROWS, CROWS), sl_hi],
                preferred_element_type=jnp.float32,
            ).astype(jnp.bfloat16)

    return pl.pallas_call(
        body,
        out_shape=jax.ShapeDtypeStruct((N_TOK, D_OUT), jnp.bfloat16),
        in_specs=[
            pl.BlockSpec(memory_space=pltpu.VMEM),
            pl.BlockSpec(memory_space=pltpu.VMEM),
            pl.BlockSpec(memory_space=pltpu.VMEM),
        ],
        out_specs=pl.BlockSpec(memory_space=pltpu.VMEM),
        scratch_shapes=[
            pltpu.VMEM((N_SLOTS, D_OUT), jnp.bfloat16),
            pltpu.SemaphoreType.DMA((N_DEV - 1,)),
            pltpu.SemaphoreType.DMA((N_DEV - 1,)),
            pltpu.SemaphoreType.DMA((N_DEV - 1,)),
            pltpu.SemaphoreType.DMA((N_DEV - 1,)),
        ],
        compiler_params=pltpu.CompilerParams(collective_id=0),
    )(x, r_col, expert_W)
